# baseline (speedup 1.0000x reference)
import functools

import jax
import jax.numpy as jnp
from jax import lax
from jax.experimental import pallas as pl
from jax.experimental.pallas import tpu as pltpu
from jax.experimental.pallas import tpu_sc as plsc

NC = 2
NS = 16
CH = 128
BROWS = 2000


def _sc_mesh():
    return plsc.VectorSubcoreMesh(core_axis_name="c", subcore_axis_name="s")


def _row_range(s, n):
    rpt = 8 * ((n + 8 * NS - 1) // (8 * NS))
    r0 = pl.multiple_of(jnp.minimum(s * rpt, n - rpt), 8)
    return r0, rpt


def _sc_degree(dst_r, ones_hb, zeros_hb, n, d):
    nch, ch = dst_r.shape[1], dst_r.shape[3]

    def body(dst_hbm, ones_hbm, zeros_hbm, out_hbm, idx_d, ones_v, accd):
        c = lax.axis_index("c")
        s = lax.axis_index("s")
        wid = c * NS + s
        r0, rpt = _row_range(s, n)
        pltpu.sync_copy(zeros_hbm.at[pl.ds(r0, rpt)], accd.at[pl.ds(r0, rpt)])
        pltpu.sync_copy(ones_hbm, ones_v)
        pltpu.sync_copy(dst_hbm.at[wid], idx_d)
        plsc.subcore_barrier()

        def step(j, carry):
            pltpu.sync_copy(ones_v, accd.at[idx_d.at[j, 0]], add=True)
            return carry

        lax.fori_loop(0, nch, step, 0)
        plsc.subcore_barrier()
        pltpu.sync_copy(accd.at[pl.ds(r0, rpt)],
                        out_hbm.at[pl.ds(c * n + r0, rpt)])

    f = pl.kernel(
        body,
        out_type=jax.ShapeDtypeStruct((NC * n, d), jnp.float32),
        mesh=_sc_mesh(),
        scratch_types=[
            pltpu.VMEM((nch, 1, ch), jnp.int32),
            pltpu.VMEM((ch, d), jnp.float32),
            pltpu.VMEM_SHARED((n + 8, d), jnp.float32),
        ],
    )
    return f(dst_r, ones_hb, zeros_hb)


def _sc_edge_agg(hp, src_r, dst_r, zerosd):
    n, d = hp.shape
    nch, ch = src_r.shape[1], src_r.shape[3]
    nhalf = 2
    hn = nch // nhalf
    assert hn % 2 == 0

    def body(hp_hbm, src_hbm, dst_hbm, zeros_hbm, out_hbm,
             idx_s, idx_d, rows, acc, sem0, sem1):
        c = lax.axis_index("c")
        s = lax.axis_index("s")
        wid = c * NS + s
        r0, rpt = _row_range(s, n)
        pltpu.sync_copy(zeros_hbm.at[pl.ds(r0, rpt)], acc.at[pl.ds(r0, rpt)])
        plsc.subcore_barrier()
        sems = (sem0, sem1)

        def gather(j, b):
            return pltpu.async_copy(hp_hbm.at[idx_s.at[j, 0]], rows.at[b],
                                    sems[b])

        def wait_gather(j, b):
            pltpu.make_async_copy(hp_hbm.at[idx_s.at[j, 0]], rows.at[b],
                                  sems[b]).wait()

        def scatter(j, b):
            pltpu.sync_copy(rows.at[b], acc.at[idx_d.at[j, 0]], add=True)

        def half(h, carry):
            h0 = pl.multiple_of(h * hn, 8)
            pltpu.sync_copy(src_hbm.at[wid].at[pl.ds(h0, hn)], idx_s)
            pltpu.sync_copy(dst_hbm.at[wid].at[pl.ds(h0, hn)], idx_d)
            gather(0, 0)
            gather(1, 1)

            def step(k, carry):
                j0 = k * 2
                j1 = j0 + 1
                wait_gather(j0, 0)
                scatter(j0, 0)

                @pl.when(j0 + 2 < hn)
                def _():
                    gather(j0 + 2, 0)

                wait_gather(j1, 1)
                scatter(j1, 1)

                @pl.when(j1 + 2 < hn)
                def _():
                    gather(j1 + 2, 1)

                return carry

            lax.fori_loop(0, hn // 2, step, 0)
            return carry

        lax.fori_loop(0, nhalf, half, 0)
        plsc.subcore_barrier()
        pltpu.sync_copy(acc.at[pl.ds(r0, rpt)],
                        out_hbm.at[pl.ds(c * n + r0, rpt)])

    f = pl.kernel(
        body,
        out_type=jax.ShapeDtypeStruct((NC * n, d), jnp.float32),
        mesh=_sc_mesh(),
        scratch_types=[
            pltpu.VMEM((hn, 1, ch), jnp.int32),
            pltpu.VMEM((hn, 1, ch), jnp.int32),
            pltpu.VMEM((2, ch, d), jnp.float32),
            pltpu.VMEM_SHARED((n + 8, d), jnp.float32),
            pltpu.SemaphoreType.DMA,
            pltpu.SemaphoreType.DMA,
        ],
    )
    return f(hp, src_r, dst_r, zerosd)


def _dinv_of(deg_blk):
    cnt = deg_blk[0] + deg_blk[1]
    return lax.rsqrt(cnt + 1.0)


def _tc_prolog(x, w, deg2):
    n, d = x.shape
    g = n // BROWS

    def body(x_ref, w_ref, deg_ref, o_ref):
        dinv = _dinv_of(deg_ref[...])
        o_ref[...] = jnp.dot(x_ref[...], w_ref[...],
                             preferred_element_type=jnp.float32) * dinv

    return pl.pallas_call(
        body,
        grid=(g,),
        in_specs=[
            pl.BlockSpec((BROWS, d), lambda i: (i, 0)),
            pl.BlockSpec((d, d), lambda i: (0, 0)),
            pl.BlockSpec((2, BROWS, 1), lambda i: (0, i, 0)),
        ],
        out_specs=pl.BlockSpec((BROWS, d), lambda i: (i, 0)),
        out_shape=jax.ShapeDtypeStruct((n, d), jnp.float32),
    )(x, w, deg2)


def _tc_combine(acc2, hp, deg2, b2d, with_stats):
    n, d = hp.shape
    g = n // BROWS

    def body(acc_ref, hp_ref, deg_ref, b_ref, o_ref, st_ref=None):
        dinv = _dinv_of(deg_ref[...])
        sval = (acc_ref[0] + acc_ref[1] + hp_ref[...]) * dinv + b_ref[...]
        o_ref[...] = sval
        if with_stats:
            st_ref[...] = jnp.concatenate(
                [jnp.sum(sval, axis=0, keepdims=True),
                 jnp.sum(sval * sval, axis=0, keepdims=True),
                 jnp.zeros((6, d), jnp.float32)], axis=0)[None]

    out_shape = [jax.ShapeDtypeStruct((n, d), jnp.float32)]
    out_specs = [pl.BlockSpec((BROWS, d), lambda i: (i, 0))]
    if with_stats:
        out_shape.append(jax.ShapeDtypeStruct((g, 8, d), jnp.float32))
        out_specs.append(pl.BlockSpec((1, 8, d), lambda i: (i, 0, 0)))

    res = pl.pallas_call(
        body,
        grid=(g,),
        in_specs=[
            pl.BlockSpec((2, BROWS, d), lambda i: (0, i, 0)),
            pl.BlockSpec((BROWS, d), lambda i: (i, 0)),
            pl.BlockSpec((2, BROWS, 1), lambda i: (0, i, 0)),
            pl.BlockSpec((1, d), lambda i: (0, 0)),
        ],
        out_specs=out_specs,
        out_shape=out_shape,
    )(acc2, hp, deg2, b2d)
    return res if with_stats else res[0]


def _tc_bn_relu_mm(out_l, st, gamma2d, beta2d, w_next, deg2, n_nodes):
    n, d = out_l.shape
    g = n // BROWS
    inv_n = 1.0 / float(n_nodes)

    def body(o_ref, st_ref, g_ref, be_ref, w_ref, deg_ref, hp_ref):
        st = st_ref[...]
        m = jnp.sum(st[:, 0, :], axis=0) * inv_n
        ex2 = jnp.sum(st[:, 1, :], axis=0) * inv_n
        v = ex2 - m * m
        dinv = _dinv_of(deg_ref[...])
        xb = (o_ref[...] - m[None, :]) * lax.rsqrt(v + 1e-5)[None, :]
        xb = xb * g_ref[...] + be_ref[...]
        x2 = jnp.maximum(xb, 0.0)
        hp_ref[...] = jnp.dot(x2, w_ref[...],
                              preferred_element_type=jnp.float32) * dinv

    return pl.pallas_call(
        body,
        grid=(g,),
        in_specs=[
            pl.BlockSpec((BROWS, d), lambda i: (i, 0)),
            pl.BlockSpec((g, 8, d), lambda i: (0, 0, 0)),
            pl.BlockSpec((1, d), lambda i: (0, 0)),
            pl.BlockSpec((1, d), lambda i: (0, 0)),
            pl.BlockSpec((d, d), lambda i: (0, 0)),
            pl.BlockSpec((2, BROWS, 1), lambda i: (0, i, 0)),
        ],
        out_specs=pl.BlockSpec((BROWS, d), lambda i: (i, 0)),
        out_shape=jax.ShapeDtypeStruct((n, d), jnp.float32),
    )(out_l, st, gamma2d, beta2d, w_next, deg2)


def kernel(x, W1, b1, gamma1, beta1, W2, b2, gamma2, beta2, W3, b3,
           gamma3, beta3, W4, b4, edge_index):
    n, d = x.shape
    e = edge_index.shape[1]
    nw = NC * NS
    t = ((e + nw * 4 * CH - 1) // (nw * 4 * CH)) * 4 * CH
    epad = t * nw
    nch = t // CH

    srcf = jnp.concatenate(
        [edge_index[0], jnp.zeros((epad - e,), edge_index.dtype)])
    dstf = jnp.concatenate(
        [edge_index[1], jnp.full((epad - e,), n, edge_index.dtype)])
    src_r = srcf.reshape(nw, nch, 1, CH)
    dst_r = dstf.reshape(nw, nch, 1, CH)
    zerosd = jnp.zeros((n, d), jnp.float32)
    onesd = jnp.ones((CH, d), jnp.float32)

    deg2 = _sc_degree(dst_r, onesd, zerosd, n, d).reshape(2, n, d)[:, :, :1]

    hp = _tc_prolog(x, W1, deg2)
    params = [(b1, gamma1, beta1, W2), (b2, gamma2, beta2, W3),
              (b3, gamma3, beta3, W4)]
    for b, gamma, beta, w_next in params:
        acc2 = _sc_edge_agg(hp, src_r, dst_r, zerosd).reshape(2, n, d)
        out_l, st = _tc_combine(acc2, hp, deg2, b.reshape(1, d), True)
        hp = _tc_bn_relu_mm(out_l, st, gamma.reshape(1, d),
                            beta.reshape(1, d), w_next, deg2, n)
    acc2 = _sc_edge_agg(hp, src_r, dst_r, zerosd).reshape(2, n, d)
    return _tc_combine(acc2, hp, deg2, b4.reshape(1, d), False)


# speedup vs baseline: 2.9524x; 2.9524x over previous
import functools

import jax
import jax.numpy as jnp
from jax import lax
from jax.experimental import pallas as pl
from jax.experimental.pallas import tpu as pltpu
from jax.experimental.pallas import tpu_sc as plsc

NC = 2
NS = 16
CH = 128
BROWS = 2000


def _sc_mesh():
    return plsc.VectorSubcoreMesh(core_axis_name="c", subcore_axis_name="s")


def _row_range(s, n):
    rpt = 8 * ((n + 8 * NS - 1) // (8 * NS))
    r0 = pl.multiple_of(jnp.minimum(s * rpt, n - rpt), 8)
    return r0, rpt


def _sc_degree(dst_r, ones_hb, zeros_hb, n, d):
    nch, ch = dst_r.shape[1], dst_r.shape[3]

    def body(dst_hbm, ones_hbm, zeros_hbm, out_hbm, idx_d, ones_v, accd):
        c = lax.axis_index("c")
        s = lax.axis_index("s")
        wid = c * NS + s
        r0, rpt = _row_range(s, n)
        pltpu.sync_copy(zeros_hbm.at[pl.ds(r0, rpt)], accd.at[pl.ds(r0, rpt)])
        pltpu.sync_copy(ones_hbm, ones_v)
        pltpu.sync_copy(dst_hbm.at[wid], idx_d)
        plsc.subcore_barrier()

        def step(j, carry):
            pltpu.sync_copy(ones_v, accd.at[idx_d.at[j, 0]], add=True)
            return carry

        lax.fori_loop(0, nch, step, 0)
        plsc.subcore_barrier()
        pltpu.sync_copy(accd.at[pl.ds(r0, rpt)],
                        out_hbm.at[pl.ds(c * n + r0, rpt)])

    f = pl.kernel(
        body,
        out_type=jax.ShapeDtypeStruct((NC * n, d), jnp.float32),
        mesh=_sc_mesh(),
        scratch_types=[
            pltpu.VMEM((nch, 1, ch), jnp.int32),
            pltpu.VMEM((ch, d), jnp.float32),
            pltpu.VMEM_SHARED((n + 8, d), jnp.float32),
        ],
    )
    return f(dst_r, ones_hb, zeros_hb)


def _sc_edge_agg(hp, src_r, dst_r, zerosd):
    n, d = hp.shape
    nch, ch = src_r.shape[1], src_r.shape[3]
    nhalf = 2
    hn = nch // nhalf
    assert hn % 2 == 0

    def body(hp_hbm, src_hbm, dst_hbm, zeros_hbm, out_hbm,
             idx_s, idx_d, rows, acc, sem0, sem1):
        c = lax.axis_index("c")
        s = lax.axis_index("s")
        wid = c * NS + s
        r0, rpt = _row_range(s, n)
        pltpu.sync_copy(zeros_hbm.at[pl.ds(r0, rpt)], acc.at[pl.ds(r0, rpt)])
        plsc.subcore_barrier()
        sems = (sem0, sem1)

        def gather(j, b):
            return pltpu.async_copy(hp_hbm.at[idx_s.at[j, 0]], rows.at[b],
                                    sems[b])

        def wait_gather(j, b):
            pltpu.make_async_copy(hp_hbm.at[idx_s.at[j, 0]], rows.at[b],
                                  sems[b]).wait()

        def scatter(j, b):
            pltpu.sync_copy(rows.at[b], acc.at[idx_d.at[j, 0]], add=True)

        def half(h, carry):
            h0 = pl.multiple_of(h * hn, 8)
            pltpu.sync_copy(src_hbm.at[wid].at[pl.ds(h0, hn)], idx_s)
            pltpu.sync_copy(dst_hbm.at[wid].at[pl.ds(h0, hn)], idx_d)
            gather(0, 0)
            gather(1, 1)

            def step(k, carry):
                j0 = k * 2
                j1 = j0 + 1
                wait_gather(j0, 0)
                scatter(j0, 0)

                @pl.when(j0 + 2 < hn)
                def _():
                    gather(j0 + 2, 0)

                wait_gather(j1, 1)
                scatter(j1, 1)

                @pl.when(j1 + 2 < hn)
                def _():
                    gather(j1 + 2, 1)

                return carry

            lax.fori_loop(0, hn // 2, step, 0)
            return carry

        lax.fori_loop(0, nhalf, half, 0)
        plsc.subcore_barrier()
        pltpu.sync_copy(acc.at[pl.ds(r0, rpt)],
                        out_hbm.at[pl.ds(c * n + r0, rpt)])

    f = pl.kernel(
        body,
        out_type=jax.ShapeDtypeStruct((NC * n, d), jnp.float32),
        mesh=_sc_mesh(),
        scratch_types=[
            pltpu.VMEM((hn, 1, ch), jnp.int32),
            pltpu.VMEM((hn, 1, ch), jnp.int32),
            pltpu.VMEM((2, ch, d), jnp.float32),
            pltpu.VMEM_SHARED((n + 8, d), jnp.float32),
            pltpu.SemaphoreType.DMA,
            pltpu.SemaphoreType.DMA,
        ],
    )
    return f(hp, src_r, dst_r, zerosd)


def _dinv_of(deg_blk):
    cnt = deg_blk[0] + deg_blk[1]
    return lax.rsqrt(cnt + 1.0)


def _tc_prolog(x, w, deg2):
    n, d = x.shape
    g = n // BROWS

    def body(x_ref, w_ref, deg_ref, o_ref):
        dinv = _dinv_of(deg_ref[...])
        o_ref[...] = jnp.dot(x_ref[...], w_ref[...],
                             preferred_element_type=jnp.float32) * dinv

    return pl.pallas_call(
        body,
        grid=(g,),
        in_specs=[
            pl.BlockSpec((BROWS, d), lambda i: (i, 0)),
            pl.BlockSpec((d, d), lambda i: (0, 0)),
            pl.BlockSpec((2, BROWS, 1), lambda i: (0, i, 0)),
        ],
        out_specs=pl.BlockSpec((BROWS, d), lambda i: (i, 0)),
        out_shape=jax.ShapeDtypeStruct((n, d), jnp.float32),
    )(x, w, deg2)


def _tc_combine(acc2, hp, deg2, b2d, with_stats):
    n, d = hp.shape
    g = n // BROWS

    def body(acc_ref, hp_ref, deg_ref, b_ref, o_ref, st_ref=None):
        dinv = _dinv_of(deg_ref[...])
        sval = (acc_ref[0] + acc_ref[1] + hp_ref[...]) * dinv + b_ref[...]
        o_ref[...] = sval
        if with_stats:
            st_ref[...] = jnp.concatenate(
                [jnp.sum(sval, axis=0, keepdims=True),
                 jnp.sum(sval * sval, axis=0, keepdims=True),
                 jnp.zeros((6, d), jnp.float32)], axis=0)[None]

    out_shape = [jax.ShapeDtypeStruct((n, d), jnp.float32)]
    out_specs = [pl.BlockSpec((BROWS, d), lambda i: (i, 0))]
    if with_stats:
        out_shape.append(jax.ShapeDtypeStruct((g, 8, d), jnp.float32))
        out_specs.append(pl.BlockSpec((1, 8, d), lambda i: (i, 0, 0)))

    res = pl.pallas_call(
        body,
        grid=(g,),
        in_specs=[
            pl.BlockSpec((2, BROWS, d), lambda i: (0, i, 0)),
            pl.BlockSpec((BROWS, d), lambda i: (i, 0)),
            pl.BlockSpec((2, BROWS, 1), lambda i: (0, i, 0)),
            pl.BlockSpec((1, d), lambda i: (0, 0)),
        ],
        out_specs=out_specs,
        out_shape=out_shape,
    )(acc2, hp, deg2, b2d)
    return res if with_stats else res[0]


def _tc_bn_relu_mm(out_l, st, gamma2d, beta2d, w_next, deg2, n_nodes):
    n, d = out_l.shape
    g = n // BROWS
    inv_n = 1.0 / float(n_nodes)

    def body(o_ref, st_ref, g_ref, be_ref, w_ref, deg_ref, hp_ref):
        st = st_ref[...]
        m = jnp.sum(st[:, 0, :], axis=0) * inv_n
        ex2 = jnp.sum(st[:, 1, :], axis=0) * inv_n
        v = ex2 - m * m
        dinv = _dinv_of(deg_ref[...])
        xb = (o_ref[...] - m[None, :]) * lax.rsqrt(v + 1e-5)[None, :]
        xb = xb * g_ref[...] + be_ref[...]
        x2 = jnp.maximum(xb, 0.0)
        hp_ref[...] = jnp.dot(x2, w_ref[...],
                              preferred_element_type=jnp.float32) * dinv

    return pl.pallas_call(
        body,
        grid=(g,),
        in_specs=[
            pl.BlockSpec((BROWS, d), lambda i: (i, 0)),
            pl.BlockSpec((g, 8, d), lambda i: (0, 0, 0)),
            pl.BlockSpec((1, d), lambda i: (0, 0)),
            pl.BlockSpec((1, d), lambda i: (0, 0)),
            pl.BlockSpec((d, d), lambda i: (0, 0)),
            pl.BlockSpec((2, BROWS, 1), lambda i: (0, i, 0)),
        ],
        out_specs=pl.BlockSpec((BROWS, d), lambda i: (i, 0)),
        out_shape=jax.ShapeDtypeStruct((n, d), jnp.float32),
    )(out_l, st, gamma2d, beta2d, w_next, deg2)


def kernel(x, W1, b1, gamma1, beta1, W2, b2, gamma2, beta2, W3, b3,
           gamma3, beta3, W4, b4, edge_index):
    n, d = x.shape
    e = edge_index.shape[1]
    nw = NC * NS
    t = ((e + nw * 4 * CH - 1) // (nw * 4 * CH)) * 4 * CH
    epad = t * nw
    nch = t // CH

    tpad = t - e // nw
    pad_src = jnp.broadcast_to(
        jnp.arange(tpad, dtype=edge_index.dtype) % n, (nw, tpad))
    src_r = jnp.concatenate(
        [edge_index[0].reshape(nw, e // nw), pad_src],
        axis=1).reshape(nw, nch, 1, CH)
    dst_r = jnp.pad(edge_index[1].reshape(nw, e // nw),
                    ((0, 0), (0, tpad)),
                    constant_values=n).reshape(nw, nch, 1, CH)
    zerosd = jnp.zeros((n, d), jnp.float32)
    onesd = jnp.ones((CH, d), jnp.float32)

    deg2 = _sc_degree(dst_r, onesd, zerosd, n, d).reshape(2, n, d)[:, :, :1]

    hp = _tc_prolog(x, W1, deg2)
    params = [(b1, gamma1, beta1, W2), (b2, gamma2, beta2, W3),
              (b3, gamma3, beta3, W4)]
    for b, gamma, beta, w_next in params:
        acc2 = _sc_edge_agg(hp, src_r, dst_r, zerosd).reshape(2, n, d)
        out_l, st = _tc_combine(acc2, hp, deg2, b.reshape(1, d), True)
        hp = _tc_bn_relu_mm(out_l, st, gamma.reshape(1, d),
                            beta.reshape(1, d), w_next, deg2, n)
    acc2 = _sc_edge_agg(hp, src_r, dst_r, zerosd).reshape(2, n, d)
    return _tc_combine(acc2, hp, deg2, b4.reshape(1, d), False)
